# baseline (speedup 1.0000x reference)
import functools

import jax
import jax.numpy as jnp
from jax import lax
from jax.experimental import pallas as pl
from jax.experimental.pallas import tpu as pltpu
from jax.experimental.pallas import tpu_sc as plsc

N_ATOMS_K = 10000
N_PAIRS_K = 160000
XDIM = 3
XPAD = 4
N_PROP_K = 128

TC_BLK = 1000
TC_GRID = N_ATOMS_K // TC_BLK

L = 16


def _bf16_bits(y):
    u = lax.bitcast_convert_type(y, jnp.uint32)
    lsb = lax.shift_right_logical(u, jnp.uint32(16)) & jnp.uint32(1)
    return lax.shift_right_logical(
        u + jnp.uint32(0x7FFF) + lsb, jnp.uint32(16))


def _proj_body(px_ref, wi_ref, wj_ref, w0_ref, w1_ref, yi_ref, yj_ref,
               a_scr, b_scr):
    @pl.when(pl.program_id(0) == 0)
    def _():
        w01 = jnp.dot(w0_ref[...], w1_ref[...],
                      preferred_element_type=jnp.float32,
                      precision=lax.Precision.HIGHEST)
        a_scr[...] = jnp.dot(wi_ref[...], w01,
                             preferred_element_type=jnp.float32,
                             precision=lax.Precision.HIGHEST)
        b_scr[...] = jnp.dot(wj_ref[...], w01,
                             preferred_element_type=jnp.float32,
                             precision=lax.Precision.HIGHEST)

    x = px_ref[...].reshape(TC_BLK * XDIM, N_PROP_K)

    def proj(scr):
        y = jnp.dot(x, scr[...], preferred_element_type=jnp.float32,
                    precision=lax.Precision.HIGHEST)
        p3 = y.reshape(TC_BLK, XDIM, N_PROP_K)
        b0 = _bf16_bits(p3[:, 0, :])
        b1 = _bf16_bits(p3[:, 1, :])
        b2 = _bf16_bits(p3[:, 2, :])
        z01 = lax.bitcast_convert_type(
            lax.shift_left(b1, jnp.uint32(16)) | b0, jnp.int32)
        z2 = lax.bitcast_convert_type(
            lax.shift_left(b2, jnp.uint32(16)) | b2, jnp.int32)
        z = jnp.concatenate(
            [z01.reshape(TC_BLK, 1, N_PROP_K),
             z2.reshape(TC_BLK, 1, N_PROP_K)], axis=1)
        return z.reshape(2 * TC_BLK, N_PROP_K)

    yi_ref[...] = proj(a_scr)
    yj_ref[...] = proj(b_scr)


@jax.jit
def _project(px, Wi, Wj, W0, W1):
    wspec = pl.BlockSpec((N_PROP_K, N_PROP_K), lambda i: (0, 0))
    return pl.pallas_call(
        _proj_body,
        grid=(TC_GRID,),
        in_specs=[
            pl.BlockSpec((TC_BLK, XDIM, N_PROP_K), lambda i: (i, 0, 0)),
            wspec, wspec, wspec, wspec,
        ],
        out_specs=[pl.BlockSpec((2 * TC_BLK, N_PROP_K),
                                lambda i: (i, 0))] * 2,
        out_shape=[jax.ShapeDtypeStruct((2 * N_ATOMS_K, N_PROP_K),
                                        jnp.int32)] * 2,
        scratch_shapes=[
            pltpu.VMEM((N_PROP_K, N_PROP_K), jnp.float32),
            pltpu.VMEM((N_PROP_K, N_PROP_K), jnp.float32),
        ],
    )(px, Wi, Wj, W0, W1)


def _make_sc_gather():
    info = plsc.get_sparse_core_info()
    nc, ns = info.num_cores, info.num_subcores
    nw = nc * ns
    per_w = N_PAIRS_K // nw
    chunk = 40
    n_real = per_w // chunk
    n_chunks = n_real + 1
    g_rows = 2 * chunk

    mesh = plsc.VectorSubcoreMesh(core_axis_name="c", subcore_axis_name="s")

    gbuf_t = pltpu.VMEM((g_rows, N_PROP_K), jnp.int32)
    sbuf_t = pltpu.VMEM((chunk, XPAD, N_PROP_K), jnp.float32)

    @functools.partial(
        pl.kernel,
        mesh=mesh,
        out_type=jax.ShapeDtypeStruct((N_PAIRS_K, XDIM, N_PROP_K),
                                      jnp.float32),
        compiler_params=pltpu.CompilerParams(needs_layout_passes=False),
        scratch_types=[
            pltpu.VMEM((n_chunks, g_rows), jnp.int32),
            pltpu.VMEM((n_chunks, g_rows), jnp.int32),
            gbuf_t, gbuf_t, gbuf_t, gbuf_t, sbuf_t, sbuf_t,
            pltpu.SemaphoreType.DMA, pltpu.SemaphoreType.DMA,
            pltpu.SemaphoreType.DMA, pltpu.SemaphoreType.DMA,
            pltpu.SemaphoreType.DMA, pltpu.SemaphoreType.DMA,
        ],
    )
    def sc_gather(yi_hbm, yj_hbm, idxi_hbm, idxj_hbm, out_hbm,
                  idxi_v, idxj_v, ga0, ga1, gb0, gb1, st0, st1,
                  gsa0, gsa1, gsb0, gsb1, sts0, sts1):
        wid = lax.axis_index("s") * nc + lax.axis_index("c")
        base = wid * per_w
        ga = (ga0, ga1)
        gb = (gb0, gb1)
        st = (st0, st1)
        gsa = (gsa0, gsa1)
        gsb = (gsb0, gsb1)
        sts = (sts0, sts1)
        pltpu.sync_copy(idxi_hbm.at[wid], idxi_v)
        pltpu.sync_copy(idxj_hbm.at[wid], idxj_v)

        def issue_gather(c, par):
            pltpu.async_copy(yi_hbm.at[idxi_v.at[c]], ga[par], gsa[par])
            pltpu.async_copy(yj_hbm.at[idxj_v.at[c]], gb[par], gsb[par])

        def wait_gather(c, par):
            pltpu.make_async_copy(
                yi_hbm.at[idxi_v.at[c]], ga[par], gsa[par]).wait()
            pltpu.make_async_copy(
                yj_hbm.at[idxj_v.at[c]], gb[par], gsb[par]).wait()

        def wait_store(c, par):
            pltpu.make_async_copy(
                st[par].at[:, pl.ds(0, XDIM)],
                out_hbm.at[pl.ds(base + c * chunk, chunk)],
                sts[par]).wait()

        issue_gather(0, 0)

        def expand(par, r, row, sl, x_lo, x_hi):
            pa = ga[par][row, sl]
            pb = gb[par][row, sl]
            sab = (plsc.bitcast(pa, jnp.bfloat16)
                   + plsc.bitcast(pb, jnp.bfloat16))
            p = plsc.bitcast(sab, jnp.int32)
            st[par][r, x_lo, sl] = plsc.bitcast(p << 16, jnp.float32)
            if x_hi is not None:
                st[par][r, x_hi, sl] = plsc.bitcast(
                    p & jnp.int32(-65536), jnp.float32)

        def step(s, carry):
            for b in range(2):
                c = 2 * s + b
                par = b
                opar = 1 - b

                @pl.when(s >= 1)
                def _():
                    wait_store(c - 2, par)

                @pl.when(c + 1 <= n_chunks - 1)
                def _():
                    issue_gather(c + 1, opar)

                wait_gather(c, par)

                @pl.when(c <= n_real - 1)
                def _():
                    def row_body(r, cr):
                        for k in range(N_PROP_K // L):
                            sl = pl.ds(k * L, L)
                            expand(par, r, 2 * r, sl, 0, 1)
                            expand(par, r, 2 * r + 1, sl, 2, None)
                        return cr

                    lax.fori_loop(0, chunk, row_body, 0)
                    pltpu.async_copy(
                        st[par].at[:, pl.ds(0, XDIM)],
                        out_hbm.at[pl.ds(base + c * chunk, chunk)],
                        sts[par])
            return carry

        lax.fori_loop(0, n_chunks // 2, step, 0)
        wait_store(n_real - 1, 0)

    return sc_gather, nw, per_w, n_chunks, chunk, g_rows


def kernel(ind_2, px, Wi, Wj, W0, W1):
    sc_gather, nw, per_w, n_chunks, chunk, g_rows = _make_sc_gather()

    yi, yj = _project(px, Wi, Wj, W0, W1)

    ind = ind_2.astype(jnp.int32)
    pad2 = n_chunks * g_rows - 2 * per_w

    def prep(col):
        a = (2 * col[:, None]
             + jnp.arange(2, dtype=jnp.int32)[None, :]).reshape(
                 nw, 2 * per_w)
        a = jnp.pad(a, ((0, 0), (0, pad2)))
        return a.reshape(nw, n_chunks, g_rows)

    idxi = prep(ind[:, 0])
    idxj = prep(ind[:, 1])

    return sc_gather(yi, yj, idxi, idxj)


# speedup vs baseline: 1.2782x; 1.2782x over previous
import functools

import jax
import jax.numpy as jnp
from jax import lax
from jax.experimental import pallas as pl
from jax.experimental.pallas import tpu as pltpu
from jax.experimental.pallas import tpu_sc as plsc

N_ATOMS_K = 10000
N_PAIRS_K = 160000
XDIM = 3
N_PROP_K = 128
ROWS = N_ATOMS_K * XDIM

TC_BLK = 1000
TC_GRID = N_ATOMS_K // TC_BLK

L = 16


def _proj_body(px_ref, wi_ref, wj_ref, w0_ref, w1_ref, yi_ref, yj_ref,
               a_scr, b_scr):
    @pl.when(pl.program_id(0) == 0)
    def _():
        w01 = jnp.dot(w0_ref[...], w1_ref[...],
                      preferred_element_type=jnp.float32,
                      precision=lax.Precision.HIGHEST)
        a_scr[...] = jnp.dot(wi_ref[...], w01,
                             preferred_element_type=jnp.float32,
                             precision=lax.Precision.HIGHEST)
        b_scr[...] = jnp.dot(wj_ref[...], w01,
                             preferred_element_type=jnp.float32,
                             precision=lax.Precision.HIGHEST)

    x = px_ref[...].reshape(TC_BLK * XDIM, N_PROP_K)
    yi_ref[...] = jnp.dot(
        x, a_scr[...], preferred_element_type=jnp.float32,
        precision=lax.Precision.HIGHEST).reshape(TC_BLK, XDIM, N_PROP_K)
    yj_ref[...] = jnp.dot(
        x, b_scr[...], preferred_element_type=jnp.float32,
        precision=lax.Precision.HIGHEST).reshape(TC_BLK, XDIM, N_PROP_K)


@jax.jit
def _project(px, Wi, Wj, W0, W1):
    wspec = pl.BlockSpec((N_PROP_K, N_PROP_K), lambda i: (0, 0))
    tspec = pl.BlockSpec((TC_BLK, XDIM, N_PROP_K), lambda i: (i, 0, 0))
    return pl.pallas_call(
        _proj_body,
        grid=(TC_GRID,),
        in_specs=[tspec, wspec, wspec, wspec, wspec],
        out_specs=[tspec, tspec],
        out_shape=[jax.ShapeDtypeStruct((N_ATOMS_K, XDIM, N_PROP_K),
                                        jnp.float32)] * 2,
        scratch_shapes=[
            pltpu.VMEM((N_PROP_K, N_PROP_K), jnp.float32),
            pltpu.VMEM((N_PROP_K, N_PROP_K), jnp.float32),
        ],
    )(px, Wi, Wj, W0, W1)


def _make_sc_gather():
    info = plsc.get_sparse_core_info()
    nc, ns = info.num_cores, info.num_subcores
    nw = nc * ns
    per_w = N_PAIRS_K // nw
    chunk = 40
    n_real = per_w // chunk
    n_chunks = n_real + 1

    mesh = plsc.VectorSubcoreMesh(core_axis_name="c", subcore_axis_name="s")

    buf_t = pltpu.VMEM((chunk, XDIM, N_PROP_K), jnp.float32)

    @functools.partial(
        pl.kernel,
        mesh=mesh,
        out_type=jax.ShapeDtypeStruct((N_PAIRS_K, XDIM, N_PROP_K),
                                      jnp.float32),
        scratch_types=[
            pltpu.VMEM((n_chunks, chunk), jnp.int32),
            pltpu.VMEM((n_chunks, chunk), jnp.int32),
            buf_t, buf_t, buf_t, buf_t,
            pltpu.SemaphoreType.DMA, pltpu.SemaphoreType.DMA,
            pltpu.SemaphoreType.DMA, pltpu.SemaphoreType.DMA,
            pltpu.SemaphoreType.DMA, pltpu.SemaphoreType.DMA,
        ],
    )
    def sc_gather(yi_hbm, yj_hbm, idxi_hbm, idxj_hbm, out_hbm,
                  idxi_v, idxj_v, ga0, ga1, gb0, gb1,
                  gsa0, gsa1, gsb0, gsb1, sts0, sts1):
        wid = lax.axis_index("s") * nc + lax.axis_index("c")
        base = wid * per_w
        ga = (ga0, ga1)
        gb = (gb0, gb1)
        gsa = (gsa0, gsa1)
        gsb = (gsb0, gsb1)
        sts = (sts0, sts1)
        pltpu.sync_copy(idxi_hbm.at[wid], idxi_v)
        pltpu.sync_copy(idxj_hbm.at[wid], idxj_v)

        half = chunk // 2

        def _gather_parts(c, par):
            return (
                (yi_hbm.at[idxi_v.at[c, pl.ds(0, half)]],
                 ga[par].at[pl.ds(0, half)], gsa[par]),
                (yi_hbm.at[idxi_v.at[c, pl.ds(half, half)]],
                 ga[par].at[pl.ds(half, half)], gsa[par]),
                (yj_hbm.at[idxj_v.at[c, pl.ds(0, half)]],
                 gb[par].at[pl.ds(0, half)], gsb[par]),
                (yj_hbm.at[idxj_v.at[c, pl.ds(half, half)]],
                 gb[par].at[pl.ds(half, half)], gsb[par]),
            )

        def issue_gather(c, par):
            for src, dst, sem in _gather_parts(c, par):
                pltpu.async_copy(src, dst, sem)

        def wait_gather(c, par):
            for src, dst, sem in _gather_parts(c, par):
                pltpu.make_async_copy(src, dst, sem).wait()

        def wait_store(c, par):
            pltpu.make_async_copy(
                ga[par], out_hbm.at[pl.ds(base + c * chunk, chunk)],
                sts[par]).wait()

        issue_gather(0, 0)

        def step(s, carry):
            for b in range(2):
                c = 2 * s + b
                par = b
                opar = 1 - b

                wait_gather(c, par)

                @pl.when(c <= n_real - 1)
                def _():
                    def row_body(r, cr):
                        for x in range(XDIM):
                            for dd in range(N_PROP_K // L):
                                sl = pl.ds(dd * L, L)
                                ga[par][r, x, sl] = (
                                    ga[par][r, x, sl] + gb[par][r, x, sl])
                        return cr

                    lax.fori_loop(0, chunk, row_body, 0)

                @pl.when(c >= 1)
                def _():
                    wait_store(c - 1, opar)

                @pl.when(c + 1 <= n_chunks - 1)
                def _():
                    issue_gather(c + 1, opar)

                @pl.when(c <= n_real - 1)
                def _():
                    pltpu.async_copy(
                        ga[par],
                        out_hbm.at[pl.ds(base + c * chunk, chunk)],
                        sts[par])
            return carry

        lax.fori_loop(0, n_chunks // 2, step, 0)

    return sc_gather, nw, per_w, n_chunks, chunk


def kernel(ind_2, px, Wi, Wj, W0, W1):
    sc_gather, nw, per_w, n_chunks, chunk = _make_sc_gather()

    yi, yj = _project(px, Wi, Wj, W0, W1)

    ind = ind_2.astype(jnp.int32)
    pad = n_chunks * chunk - per_w

    def prep(col):
        a = col.reshape(nw, per_w)
        a = jnp.pad(a, ((0, 0), (0, pad)))
        return a.reshape(nw, n_chunks, chunk)

    idxi = prep(ind[:, 0])
    idxj = prep(ind[:, 1])

    return sc_gather(yi, yj, idxi, idxj)


# speedup vs baseline: 1.3443x; 1.0517x over previous
import functools

import jax
import jax.numpy as jnp
from jax import lax
from jax.experimental import pallas as pl
from jax.experimental.pallas import tpu as pltpu
from jax.experimental.pallas import tpu_sc as plsc

N_ATOMS_K = 10000
N_PAIRS_K = 160000
XDIM = 3
N_PROP_K = 128
ROWS = N_ATOMS_K * XDIM

TC_BLK = 1000
TC_GRID = N_ATOMS_K // TC_BLK

L = 16


def _proj_body(px_ref, wi_ref, wj_ref, w0_ref, w1_ref, yi_ref, yj_ref,
               a_scr, b_scr):
    @pl.when(pl.program_id(0) == 0)
    def _():
        w01 = jnp.dot(w0_ref[...], w1_ref[...],
                      preferred_element_type=jnp.float32,
                      precision=lax.Precision.HIGHEST)
        a_scr[...] = jnp.dot(wi_ref[...], w01,
                             preferred_element_type=jnp.float32,
                             precision=lax.Precision.HIGHEST)
        b_scr[...] = jnp.dot(wj_ref[...], w01,
                             preferred_element_type=jnp.float32,
                             precision=lax.Precision.HIGHEST)

    x = px_ref[...].reshape(TC_BLK * XDIM, N_PROP_K)
    yi_ref[...] = jnp.dot(
        x, a_scr[...],
        preferred_element_type=jnp.float32).reshape(TC_BLK, XDIM, N_PROP_K)
    yj_ref[...] = jnp.dot(
        x, b_scr[...],
        preferred_element_type=jnp.float32).reshape(TC_BLK, XDIM, N_PROP_K)


@jax.jit
def _project(px, Wi, Wj, W0, W1):
    wspec = pl.BlockSpec((N_PROP_K, N_PROP_K), lambda i: (0, 0))
    tspec = pl.BlockSpec((TC_BLK, XDIM, N_PROP_K), lambda i: (i, 0, 0))
    return pl.pallas_call(
        _proj_body,
        grid=(TC_GRID,),
        in_specs=[tspec, wspec, wspec, wspec, wspec],
        out_specs=[tspec, tspec],
        out_shape=[jax.ShapeDtypeStruct((N_ATOMS_K, XDIM, N_PROP_K),
                                        jnp.float32)] * 2,
        scratch_shapes=[
            pltpu.VMEM((N_PROP_K, N_PROP_K), jnp.float32),
            pltpu.VMEM((N_PROP_K, N_PROP_K), jnp.float32),
        ],
    )(px, Wi, Wj, W0, W1)


def _make_sc_gather():
    info = plsc.get_sparse_core_info()
    nc, ns = info.num_cores, info.num_subcores
    nw = nc * ns
    per_w = N_PAIRS_K // nw
    chunk = 40
    n_real = per_w // chunk
    n_chunks = n_real + 1

    mesh = plsc.VectorSubcoreMesh(core_axis_name="c", subcore_axis_name="s")

    buf_t = pltpu.VMEM((chunk, XDIM, N_PROP_K), jnp.float32)

    @functools.partial(
        pl.kernel,
        mesh=mesh,
        out_type=jax.ShapeDtypeStruct((N_PAIRS_K, XDIM, N_PROP_K),
                                      jnp.float32),
        scratch_types=[
            pltpu.VMEM((n_chunks, chunk), jnp.int32),
            pltpu.VMEM((n_chunks, chunk), jnp.int32),
            buf_t, buf_t, buf_t, buf_t,
            pltpu.SemaphoreType.DMA, pltpu.SemaphoreType.DMA,
            pltpu.SemaphoreType.DMA, pltpu.SemaphoreType.DMA,
            pltpu.SemaphoreType.DMA, pltpu.SemaphoreType.DMA,
        ],
    )
    def sc_gather(yi_hbm, yj_hbm, idxi_hbm, idxj_hbm, out_hbm,
                  idxi_v, idxj_v, ga0, ga1, gb0, gb1,
                  gsa0, gsa1, gsb0, gsb1, sts0, sts1):
        wid = lax.axis_index("s") * nc + lax.axis_index("c")
        base = wid * per_w
        ga = (ga0, ga1)
        gb = (gb0, gb1)
        gsa = (gsa0, gsa1)
        gsb = (gsb0, gsb1)
        sts = (sts0, sts1)
        pltpu.sync_copy(idxi_hbm.at[wid], idxi_v)
        pltpu.sync_copy(idxj_hbm.at[wid], idxj_v)

        half = chunk // 2

        def _gather_parts(c, par):
            return (
                (yi_hbm.at[idxi_v.at[c, pl.ds(0, half)]],
                 ga[par].at[pl.ds(0, half)], gsa[par]),
                (yi_hbm.at[idxi_v.at[c, pl.ds(half, half)]],
                 ga[par].at[pl.ds(half, half)], gsa[par]),
                (yj_hbm.at[idxj_v.at[c, pl.ds(0, half)]],
                 gb[par].at[pl.ds(0, half)], gsb[par]),
                (yj_hbm.at[idxj_v.at[c, pl.ds(half, half)]],
                 gb[par].at[pl.ds(half, half)], gsb[par]),
            )

        def issue_gather(c, par):
            for src, dst, sem in _gather_parts(c, par):
                pltpu.async_copy(src, dst, sem)

        def wait_gather(c, par):
            for src, dst, sem in _gather_parts(c, par):
                pltpu.make_async_copy(src, dst, sem).wait()

        def wait_store(c, par):
            pltpu.make_async_copy(
                ga[par], out_hbm.at[pl.ds(base + c * chunk, chunk)],
                sts[par]).wait()

        issue_gather(0, 0)

        def step(s, carry):
            for b in range(2):
                c = 2 * s + b
                par = b
                opar = 1 - b

                wait_gather(c, par)

                @pl.when(c <= n_real - 1)
                def _():
                    def row_body(r, cr):
                        for x in range(XDIM):
                            for dd in range(N_PROP_K // L):
                                sl = pl.ds(dd * L, L)
                                ga[par][r, x, sl] = (
                                    ga[par][r, x, sl] + gb[par][r, x, sl])
                        return cr

                    lax.fori_loop(0, chunk, row_body, 0)

                @pl.when(c >= 1)
                def _():
                    wait_store(c - 1, opar)

                @pl.when(c + 1 <= n_chunks - 1)
                def _():
                    issue_gather(c + 1, opar)

                @pl.when(c <= n_real - 1)
                def _():
                    pltpu.async_copy(
                        ga[par],
                        out_hbm.at[pl.ds(base + c * chunk, chunk)],
                        sts[par])
            return carry

        lax.fori_loop(0, n_chunks // 2, step, 0)

    return sc_gather, nw, per_w, n_chunks, chunk


def kernel(ind_2, px, Wi, Wj, W0, W1):
    sc_gather, nw, per_w, n_chunks, chunk = _make_sc_gather()

    yi, yj = _project(px, Wi, Wj, W0, W1)

    ind = ind_2.astype(jnp.int32)
    pad = n_chunks * chunk - per_w

    def prep(col):
        a = col.reshape(nw, per_w)
        a = jnp.pad(a, ((0, 0), (0, pad)))
        return a.reshape(nw, n_chunks, chunk)

    idxi = prep(ind[:, 0])
    idxj = prep(ind[:, 1])

    return sc_gather(yi, yj, idxi, idxj)
